# baseline (speedup 1.0000x reference)
import functools

import numpy as np
import jax
import jax.numpy as jnp
from jax import lax
from jax.experimental import pallas as pl
from jax.experimental.pallas import tpu as pltpu
from jax.experimental.pallas import tpu_sc as plsc

ALPHA = 0.1
THETA = 0.5

NTILE = 16
NCORE = 2
CH = 128
FH = 64


def _rsqrt16(x):
    i = plsc.bitcast(x, jnp.int32)
    i = jnp.int32(0x5F3759DF) - lax.shift_right_logical(i, 1)
    y = plsc.bitcast(i, jnp.float32)
    for _ in range(3):
        y = y * (1.5 - 0.5 * x * y * y)
    return y


def _lin0_body(x_ref, w_ref, b_ref, o_ref):
    acc = jnp.dot(x_ref[...], w_ref[...], preferred_element_type=jnp.float32)
    o_ref[...] = jnp.maximum(acc + b_ref[...], 0.0).astype(jnp.bfloat16)


def _lin0(x, W0, b0):
    n, in_f = x.shape
    hid = W0.shape[1]
    bn = 2000
    grid = n // bn
    return pl.pallas_call(
        _lin0_body,
        grid=(grid,),
        in_specs=[
            pl.BlockSpec((bn, in_f), lambda i: (i, 0)),
            pl.BlockSpec((in_f, hid), lambda i: (0, 0)),
            pl.BlockSpec((1, hid), lambda i: (0, 0)),
        ],
        out_specs=pl.BlockSpec((bn, hid), lambda i: (i, 0)),
        out_shape=jax.ShapeDtypeStruct((n, hid), jnp.bfloat16),
    )(x, W0, b0.reshape(1, hid))


def _make_mix(beta, last):
    a1 = 1.0 - ALPHA
    a0 = ALPHA
    b1 = 1.0 - beta
    b0c = beta

    def tmix(p_ref, perm_ref, x0_ref, w_ref):
        ps = jnp.dot(p_ref[...], perm_ref[...],
                     preferred_element_type=jnp.float32)
        t = a1 * ps + a0 * x0_ref[...].astype(jnp.float32)
        return b1 * t + b0c * jnp.dot(t, w_ref[...],
                                      preferred_element_type=jnp.float32)

    if last:
        def body(p_ref, perm_ref, x0_ref, w_ref, wo_ref, bo_ref, o_ref):
            r = jnp.maximum(tmix(p_ref, perm_ref, x0_ref, w_ref), 0.0)
            o_ref[...] = jnp.dot(r, wo_ref[...],
                                 preferred_element_type=jnp.float32) + bo_ref[...]
    else:
        def body(p_ref, perm_ref, x0_ref, w_ref, o_ref):
            r = jnp.maximum(tmix(p_ref, perm_ref, x0_ref, w_ref), 0.0)
            o_ref[...] = r.astype(jnp.bfloat16)
    return body


def _tc_layer(p, perm, x0, Wl, beta, last, W_out=None, b_out=None):
    e, h2 = p.shape
    bn = 1280
    grid = e // bn
    out_f = W_out.shape[1] if last else h2
    blk = lambda i: (i, 0)
    zero = lambda i: (0, 0)
    in_specs = [
        pl.BlockSpec((bn, h2), blk),
        pl.BlockSpec((h2, h2), zero),
        pl.BlockSpec((bn, h2), blk),
        pl.BlockSpec((h2, h2), zero),
    ]
    args = [p, perm, x0, Wl]
    if last:
        in_specs += [pl.BlockSpec((h2, out_f), zero),
                     pl.BlockSpec((1, out_f), zero)]
        args += [W_out, b_out.reshape(1, out_f)]
    return pl.pallas_call(
        _make_mix(beta, last),
        grid=(grid,),
        in_specs=in_specs,
        out_specs=pl.BlockSpec((bn, out_f), blk),
        out_shape=jax.ShapeDtypeStruct(
            (e, out_f), jnp.float32 if last else jnp.bfloat16),
    )(*args)


def _sc_setup(h, ef, n, e):
    ept = e // NTILE
    nfull = ept // CH
    rem = ept - nfull * CH
    npt = n // NTILE
    hid = h.shape[1]

    mesh = plsc.VectorSubcoreMesh(core_axis_name="c", subcore_axis_name="s")

    @functools.partial(
        pl.kernel,
        mesh=mesh,
        compiler_params=pltpu.CompilerParams(use_tc_tiling_on_sc=False, needs_layout_passes=False),
        out_type=[
            jax.ShapeDtypeStruct((e, 2 * hid), jnp.bfloat16),
            jax.ShapeDtypeStruct((e,), jnp.float32),
        ],
        scratch_types=[
            pltpu.VMEM_SHARED((n, hid), jnp.bfloat16),
            pltpu.VMEM_SHARED((n, 16), jnp.float32),
            pltpu.VMEM((npt, hid), jnp.bfloat16),
            pltpu.VMEM((npt, 16), jnp.float32),
            pltpu.VMEM((CH, 16), jnp.float32),
            pltpu.VMEM((CH,), jnp.int32),
            pltpu.VMEM((16,), jnp.int32),
            pltpu.VMEM((CH, 16), jnp.float32),
            pltpu.VMEM((ept,), jnp.float32),
            pltpu.VMEM((CH, hid), jnp.bfloat16),
        ],
    )
    def setup(h_hbm, ef_hbm, lx_hbm, dis_hbm, h_sp, deg_sp, stage, zbuf,
              ones_v, idx128, idxr, dtmp, dis_all, grow):
        core = lax.axis_index("c")
        tid = lax.axis_index("s")
        ebase = tid * ept

        @pl.loop(0, npt)
        def _z(r):
            zbuf[r] = jnp.zeros((16,), jnp.float32)

        pltpu.sync_copy(zbuf, deg_sp.at[pl.ds(tid * npt, npt)])
        pltpu.sync_copy(h_hbm.at[pl.ds(tid * npt, npt)], stage)
        pltpu.sync_copy(stage, h_sp.at[pl.ds(tid * npt, npt)])
        plsc.subcore_barrier()

        @pl.loop(0, CH)
        def _o(r):
            ones_v[r] = jnp.ones((16,), jnp.float32)

        @pl.loop(0, nfull)
        def _deg(k):
            pltpu.sync_copy(ef_hbm.at[pl.ds(e + ebase + k * CH, CH)], idx128)
            pltpu.sync_copy(ones_v, deg_sp.at[idx128], add=True)

        if rem:
            pltpu.sync_copy(ef_hbm.at[pl.ds(e + ebase + nfull * CH, rem)], idxr)
            pltpu.sync_copy(ones_v.at[pl.ds(0, rem)], deg_sp.at[idxr], add=True)
        plsc.subcore_barrier()

        lanes = jax.lax.iota(jnp.int32, 16)
        zeros16 = jnp.zeros((16,), jnp.int32)

        @pl.loop(0, nfull)
        def _dis(k):
            pltpu.sync_copy(ef_hbm.at[pl.ds(ebase + k * CH, CH)], idx128)
            pltpu.sync_copy(deg_sp.at[idx128], dtmp)
            for j in range(CH // 16):
                d = plsc.load_gather(dtmp, [j * 16 + lanes, zeros16])
                dis_all[pl.ds(k * CH + j * 16, 16)] = _rsqrt16(d + 1.0)

        if rem:
            pltpu.sync_copy(ef_hbm.at[pl.ds(ebase + nfull * CH, rem)], idxr)
            pltpu.sync_copy(deg_sp.at[idxr], dtmp.at[pl.ds(0, rem)])
            for j in range(rem // 16):
                d = plsc.load_gather(dtmp, [j * 16 + lanes, zeros16])
                dis_all[pl.ds(nfull * CH + j * 16, 16)] = _rsqrt16(d + 1.0)

        @pl.when(core == 0)
        def _():
            pltpu.sync_copy(dis_all, dis_hbm.at[pl.ds(ebase, ept)])

        @pl.loop(0, nfull)
        def _lx(k):
            pltpu.sync_copy(ef_hbm.at[pl.ds(core * e + ebase + k * CH, CH)],
                            idx128)
            pltpu.sync_copy(h_sp.at[idx128], grow)
            pltpu.sync_copy(grow,
                            lx_hbm.at[pl.ds(ebase + k * CH, CH),
                                      pl.ds(core * hid, hid)])

        if rem:
            pltpu.sync_copy(ef_hbm.at[pl.ds(core * e + ebase + nfull * CH,
                                            rem)], idxr)
            pltpu.sync_copy(h_sp.at[idxr], grow.at[pl.ds(0, rem)])
            pltpu.sync_copy(grow.at[pl.ds(0, rem)],
                            lx_hbm.at[pl.ds(ebase + nfull * CH, rem),
                                      pl.ds(core * hid, hid)])

    return setup(h, ef)


FW = FH // 2


def _sc_prop(cur, dis, ef, n, e):
    ept = e // NTILE
    nfull = ept // CH
    rem = ept - nfull * CH
    npt = n // NTILE
    assert nfull % 2 == 0 and nfull >= 4
    h2 = cur.shape[1]

    mesh = plsc.VectorSubcoreMesh(core_axis_name="c", subcore_axis_name="s")

    @functools.partial(
        pl.kernel,
        mesh=mesh,
        compiler_params=pltpu.CompilerParams(use_tc_tiling_on_sc=False,
                                             needs_layout_passes=False),
        out_type=jax.ShapeDtypeStruct((e, h2), jnp.float32),
        scratch_types=[
            pltpu.VMEM_SHARED((n, FH), jnp.float32),
            pltpu.VMEM((npt, FH), jnp.float32),
            pltpu.VMEM((CH, FH), jnp.bfloat16),
            pltpu.VMEM((CH, FH), jnp.bfloat16),
            pltpu.VMEM((CH, FH), jnp.float32),
            pltpu.VMEM((CH, FH), jnp.float32),
            pltpu.VMEM((CH, FH), jnp.float32),
            pltpu.VMEM((CH, FH), jnp.float32),
            pltpu.VMEM((CH,), jnp.float32),
            pltpu.VMEM((CH,), jnp.float32),
            pltpu.VMEM((CH,), jnp.int32),
            pltpu.VMEM((CH,), jnp.int32),
            pltpu.VMEM((16,), jnp.int32),
            pltpu.SemaphoreType.DMA,
            pltpu.SemaphoreType.DMA,
            pltpu.SemaphoreType.DMA,
            pltpu.SemaphoreType.DMA,
        ],
    )
    def prop(cur_hbm, dis_hbm, ef_hbm, p_hbm, agg_sp, zstage, z0, z1, u_f32,
             p0, p1, g_v, d0, d1, i0, i1, idxr, lsem0, lsem1, osem0, osem1):
        core = lax.axis_index("c")
        tid = lax.axis_index("s")
        ebase = tid * ept
        fbase = core * FH
        zvs, pvs, dvs, ivs = [z0, z1], [p0, p1], [d0, d1], [i0, i1]
        lsems, osems = [lsem0, lsem1], [osem0, osem1]
        himask = jnp.full((16,), -65536, jnp.int32)

        @pl.loop(0, npt)
        def _z(r):
            for f in range(FH // 16):
                zstage[r, pl.ds(f * 16, 16)] = jnp.zeros((16,), jnp.float32)

        pltpu.sync_copy(zstage, agg_sp.at[pl.ds(tid * npt, npt)])
        plsc.subcore_barrier()

        def load_descs(k, b, use_dst):
            base = ebase + k * CH
            ioff = e + base if use_dst else base
            return (
                pltpu.make_async_copy(ef_hbm.at[pl.ds(ioff, CH)], ivs[b],
                                      lsems[b]),
                pltpu.make_async_copy(dis_hbm.at[pl.ds(base, CH)], dvs[b],
                                      lsems[b]),
                pltpu.make_async_copy(
                    cur_hbm.at[pl.ds(base, CH), pl.ds(fbase, FH)], zvs[b],
                    lsems[b]),
            )

        def extract(z_v, i, g2):
            w = plsc.bitcast(z_v[i, pl.ds(g2 * 32, 32)], jnp.int32)
            a = plsc.bitcast(lax.shift_left(w, 16), jnp.float32)
            bb = plsc.bitcast(jnp.bitwise_and(w, himask), jnp.float32)
            return a, bb

        def scale_row(z_v, i, s):
            for g2 in range(FW // 16):
                a, bb = extract(z_v, i, g2)
                u_f32[i, pl.ds(g2 * 32, 16)] = s * a
                u_f32[i, pl.ds(g2 * 32 + 16, 16)] = s * bb

        def combine_row(z_v, i, s, dst):
            s2 = s * s
            for g2 in range(FW // 16):
                a, bb = extract(z_v, i, g2)
                sl0 = pl.ds(g2 * 32, 16)
                sl1 = pl.ds(g2 * 32 + 16, 16)
                dst[i, sl0] = s * g_v[i, sl0] + s2 * a
                dst[i, sl1] = s * g_v[i, sl1] + s2 * bb

        def scale_chunk(b, nrows):
            z_v, d_v = zvs[b], dvs[b]

            @pl.loop(0, nrows // 16)
            def _s(g):
                dvec = d_v[pl.ds(g * 16, 16)]
                for j in range(16):
                    scale_row(z_v, g * 16 + j, dvec[j])

        def combine_chunk(b, nrows, dst):
            z_v, d_v = zvs[b], dvs[b]

            @pl.loop(0, nrows // 16)
            def _c(g):
                dvec = d_v[pl.ds(g * 16, 16)]
                for j in range(16):
                    combine_row(z_v, g * 16 + j, dvec[j], dst)

        for b in range(2):
            for dsc in load_descs(b, b, True):
                dsc.start()

        @pl.loop(0, nfull // 2)
        def _scat(gi):
            for b in range(2):
                k = gi * 2 + b
                for dsc in load_descs(k, b, True):
                    dsc.wait()
                scale_chunk(b, CH)
                pltpu.sync_copy(u_f32, agg_sp.at[ivs[b]], add=True)

                @pl.when(k + 2 < nfull)
                def _():
                    for dsc in load_descs(k + 2, b, True):
                        dsc.start()

        if rem:
            base = ebase + nfull * CH
            pltpu.sync_copy(ef_hbm.at[pl.ds(e + base, rem)], idxr)
            pltpu.sync_copy(dis_hbm.at[pl.ds(base, rem)], d0.at[pl.ds(0, rem)])
            pltpu.sync_copy(cur_hbm.at[pl.ds(base, rem), pl.ds(fbase, FH)],
                            z0.at[pl.ds(0, rem)])

            @pl.loop(0, rem // 16)
            def _sr(g):
                dvec = d0[pl.ds(g * 16, 16)]
                for j in range(16):
                    scale_row(z0, g * 16 + j, dvec[j])

            pltpu.sync_copy(u_f32.at[pl.ds(0, rem)], agg_sp.at[idxr], add=True)
        plsc.subcore_barrier()

        def out_desc(k, b):
            base = ebase + k * CH
            return pltpu.make_async_copy(
                pvs[b], p_hbm.at[pl.ds(base, CH), pl.ds(fbase, FH)], osems[b])

        for b in range(2):
            for dsc in load_descs(b, b, False):
                dsc.start()

        @pl.loop(0, nfull // 2)
        def _gath(gi):
            for b in range(2):
                k = gi * 2 + b
                for dsc in load_descs(k, b, False):
                    dsc.wait()
                pltpu.sync_copy(agg_sp.at[ivs[b]], g_v)

                @pl.when(gi >= 1)
                def _():
                    out_desc(k, b).wait()

                combine_chunk(b, CH, pvs[b])
                out_desc(k, b).start()

                @pl.when(k + 2 < nfull)
                def _():
                    for dsc in load_descs(k + 2, b, False):
                        dsc.start()

        for b in range(2):
            out_desc(nfull - 2 + b, b).wait()

        if rem:
            base = ebase + nfull * CH
            pltpu.sync_copy(ef_hbm.at[pl.ds(base, rem)], idxr)
            pltpu.sync_copy(dis_hbm.at[pl.ds(base, rem)], d0.at[pl.ds(0, rem)])
            pltpu.sync_copy(cur_hbm.at[pl.ds(base, rem), pl.ds(fbase, FH)],
                            z0.at[pl.ds(0, rem)])
            pltpu.sync_copy(agg_sp.at[idxr], g_v.at[pl.ds(0, rem)])

            @pl.loop(0, rem // 16)
            def _cr(g):
                dvec = d0[pl.ds(g * 16, 16)]
                for j in range(16):
                    combine_row(z0, g * 16 + j, dvec[j], p0)

            pltpu.sync_copy(p0.at[pl.ds(0, rem)],
                            p_hbm.at[pl.ds(base, rem), pl.ds(fbase, FH)])

    return prop(cur, dis, ef)


def kernel(x, edge_index, W0, b0, conv_W, W_out, b_out):
    n = x.shape[0]
    e = edge_index.shape[1]
    num_layers = conv_W.shape[0]

    ef = edge_index.reshape(-1).astype(jnp.int32)

    h = _lin0(x, W0, b0)
    lx, dis = _sc_setup(h, ef, n, e)

    h2 = 2 * W0.shape[1]
    cols = np.arange(h2)
    gg, rr = cols // 32, cols % 32
    blocked = gg * 32 + rr // 2 + 16 * (rr % 2)
    pmat = np.zeros((h2, h2), np.float32)
    pmat[blocked, cols] = 1.0
    perm = jnp.asarray(pmat)

    cur = lx
    out = None
    for l in range(num_layers):
        beta = float(np.log(THETA / (l + 1) + 1.0))
        p = _sc_prop(cur, dis, ef, n, e)
        last = l == num_layers - 1
        if last:
            out = _tc_layer(p, perm, lx, conv_W[l], beta, True, W_out, b_out)
        else:
            cur = _tc_layer(p, perm, lx, conv_W[l], beta, False)
    return out


# speedup vs baseline: 1.8386x; 1.8386x over previous
import functools

import numpy as np
import jax
import jax.numpy as jnp
from jax import lax
from jax.experimental import pallas as pl
from jax.experimental.pallas import tpu as pltpu
from jax.experimental.pallas import tpu_sc as plsc

ALPHA = 0.1
THETA = 0.5

NTILE = 16
CH = 128
FH = 64

_SC_PARAMS = pltpu.CompilerParams(use_tc_tiling_on_sc=False,
                                  needs_layout_passes=False)


def _rsqrt16(x):
    i = plsc.bitcast(x, jnp.int32)
    i = jnp.int32(0x5F3759DF) - lax.shift_right_logical(i, 1)
    y = plsc.bitcast(i, jnp.float32)
    for _ in range(3):
        y = y * (1.5 - 0.5 * x * y * y)
    return y


def _lin0_body(x_ref, w_ref, b_ref, o_ref):
    acc = jnp.dot(x_ref[...], w_ref[...], preferred_element_type=jnp.float32)
    o_ref[...] = jnp.maximum(acc + b_ref[...], 0.0)


def _lin0(x, W0, b0):
    n, in_f = x.shape
    hid = W0.shape[1]
    bn = 2000
    grid = n // bn
    return pl.pallas_call(
        _lin0_body,
        grid=(grid,),
        in_specs=[
            pl.BlockSpec((bn, in_f), lambda i: (i, 0)),
            pl.BlockSpec((in_f, hid), lambda i: (0, 0)),
            pl.BlockSpec((1, hid), lambda i: (0, 0)),
        ],
        out_specs=pl.BlockSpec((bn, hid), lambda i: (i, 0)),
        out_shape=jax.ShapeDtypeStruct((n, hid), jnp.float32),
    )(x, W0, b0.reshape(1, hid))


def _make_mix(beta, last):
    a1 = 1.0 - ALPHA
    a0 = ALPHA
    b1 = 1.0 - beta
    b0c = beta

    def tmix(p_ref, x0_ref, w_ref):
        t = a1 * p_ref[...] + a0 * x0_ref[...]
        return b1 * t + b0c * jnp.dot(t, w_ref[...],
                                      preferred_element_type=jnp.float32)

    if last:
        def body(p_ref, x0_ref, w_ref, wo_ref, bo_ref, o_ref):
            r = jnp.maximum(tmix(p_ref, x0_ref, w_ref), 0.0)
            o_ref[...] = jnp.dot(r, wo_ref[...],
                                 preferred_element_type=jnp.float32) + bo_ref[...]
    else:
        def body(p_ref, x0_ref, w_ref, o_ref):
            r = jnp.maximum(tmix(p_ref, x0_ref, w_ref), 0.0)
            o_ref[...] = r
    return body


def _tc_layer(p, x0, Wl, beta, last, W_out=None, b_out=None):
    e, h2 = p.shape
    bn = 1280
    grid = e // bn
    out_f = W_out.shape[1] if last else h2
    blk = lambda i: (i, 0)
    zero = lambda i: (0, 0)
    in_specs = [
        pl.BlockSpec((bn, h2), blk),
        pl.BlockSpec((bn, h2), blk),
        pl.BlockSpec((h2, h2), zero),
    ]
    args = [p, x0, Wl]
    if last:
        in_specs += [pl.BlockSpec((h2, out_f), zero),
                     pl.BlockSpec((1, out_f), zero)]
        args += [W_out, b_out.reshape(1, out_f)]
    return pl.pallas_call(
        _make_mix(beta, last),
        grid=(grid,),
        in_specs=in_specs,
        out_specs=pl.BlockSpec((bn, out_f), blk),
        out_shape=jax.ShapeDtypeStruct((e, out_f), jnp.float32),
    )(*args)


def _sc_setup(h, ef, n, e):
    ept = e // NTILE
    nfull = ept // CH
    rem = ept - nfull * CH
    npt = n // NTILE
    hid = h.shape[1]
    assert nfull % 2 == 0 and nfull >= 4

    mesh = plsc.VectorSubcoreMesh(core_axis_name="c", subcore_axis_name="s")

    @functools.partial(
        pl.kernel,
        mesh=mesh,
        compiler_params=_SC_PARAMS,
        out_type=[
            jax.ShapeDtypeStruct((e, 2 * hid), jnp.float32),
            jax.ShapeDtypeStruct((e,), jnp.float32),
        ],
        scratch_types=[
            pltpu.VMEM_SHARED((n, hid), jnp.float32),
            pltpu.VMEM_SHARED((n, 16), jnp.float32),
            pltpu.VMEM((npt, hid), jnp.float32),
            pltpu.VMEM((CH, 16), jnp.float32),
            pltpu.VMEM((CH,), jnp.int32),
            pltpu.VMEM((CH,), jnp.int32),
            pltpu.VMEM((16,), jnp.int32),
            pltpu.VMEM((CH, 16), jnp.float32),
            pltpu.VMEM((ept,), jnp.float32),
            pltpu.VMEM((CH, hid), jnp.float32),
            pltpu.VMEM((CH, hid), jnp.float32),
            pltpu.SemaphoreType.DMA,
            pltpu.SemaphoreType.DMA,
            pltpu.SemaphoreType.DMA,
            pltpu.SemaphoreType.DMA,
        ],
    )
    def setup(h_hbm, ef_hbm, lx_hbm, dis_hbm, h_sp, deg_sp, stage, ones_v,
              i0, i1, idxr, dtmp, dis_all, g0, g1,
              lsem0, lsem1, osem0, osem1):
        core = lax.axis_index("c")
        tid = lax.axis_index("s")
        ebase = tid * ept
        ivs, gvs = [i0, i1], [g0, g1]
        lsems, osems = [lsem0, lsem1], [osem0, osem1]

        @pl.loop(0, npt)
        def _z(r):
            for f in range(hid // 16):
                stage[r, pl.ds(f * 16, 16)] = jnp.zeros((16,), jnp.float32)

        pltpu.sync_copy(stage.at[:, pl.ds(0, 16)],
                        deg_sp.at[pl.ds(tid * npt, npt)])
        pltpu.sync_copy(h_hbm.at[pl.ds(tid * npt, npt)], stage)
        pltpu.sync_copy(stage, h_sp.at[pl.ds(tid * npt, npt)])
        plsc.subcore_barrier()

        @pl.loop(0, CH)
        def _o(r):
            ones_v[r] = jnp.ones((16,), jnp.float32)

        def idx_desc(k, b, which):
            return pltpu.make_async_copy(
                ef_hbm.at[pl.ds(which * e + ebase + k * CH, CH)], ivs[b],
                lsems[b])

        for b in range(2):
            idx_desc(b, b, 1).start()

        @pl.loop(0, nfull // 2)
        def _deg(gi):
            for b in range(2):
                k = gi * 2 + b
                idx_desc(k, b, 1).wait()
                pltpu.sync_copy(ones_v, deg_sp.at[ivs[b]], add=True)

                @pl.when(k + 2 < nfull)
                def _():
                    idx_desc(k + 2, b, 1).start()

        if rem:
            pltpu.sync_copy(ef_hbm.at[pl.ds(e + ebase + nfull * CH, rem)],
                            idxr)
            pltpu.sync_copy(ones_v.at[pl.ds(0, rem)], deg_sp.at[idxr],
                            add=True)
        plsc.subcore_barrier()

        lanes = lax.iota(jnp.int32, 16)
        zeros16 = jnp.zeros((16,), jnp.int32)

        for b in range(2):
            idx_desc(b, b, 0).start()

        @pl.loop(0, nfull // 2)
        def _dis(gi):
            for b in range(2):
                k = gi * 2 + b
                idx_desc(k, b, 0).wait()
                pltpu.sync_copy(deg_sp.at[ivs[b]], dtmp)
                for j in range(CH // 16):
                    d = plsc.load_gather(dtmp, [j * 16 + lanes, zeros16])
                    dis_all[pl.ds(k * CH + j * 16, 16)] = _rsqrt16(d + 1.0)

                @pl.when(k + 2 < nfull)
                def _():
                    idx_desc(k + 2, b, 0).start()

        if rem:
            pltpu.sync_copy(ef_hbm.at[pl.ds(ebase + nfull * CH, rem)], idxr)
            pltpu.sync_copy(deg_sp.at[idxr], dtmp.at[pl.ds(0, rem)])
            for j in range(rem // 16):
                d = plsc.load_gather(dtmp, [j * 16 + lanes, zeros16])
                dis_all[pl.ds(nfull * CH + j * 16, 16)] = _rsqrt16(d + 1.0)

        @pl.when(core == 0)
        def _():
            pltpu.sync_copy(dis_all, dis_hbm.at[pl.ds(ebase, ept)])

        def out_desc(k, b):
            return pltpu.make_async_copy(
                gvs[b],
                lx_hbm.at[pl.ds(ebase + k * CH, CH), pl.ds(core * hid, hid)],
                osems[b])

        for b in range(2):
            idx_desc(b, b, core).start()

        @pl.loop(0, nfull // 2)
        def _lx(gi):
            for b in range(2):
                k = gi * 2 + b
                idx_desc(k, b, core).wait()

                @pl.when(gi >= 1)
                def _():
                    out_desc(k, b).wait()

                pltpu.sync_copy(h_sp.at[ivs[b]], gvs[b])
                out_desc(k, b).start()

                @pl.when(k + 2 < nfull)
                def _():
                    idx_desc(k + 2, b, core).start()

        for b in range(2):
            out_desc(nfull - 2 + b, b).wait()

        if rem:
            pltpu.sync_copy(ef_hbm.at[pl.ds(core * e + ebase + nfull * CH,
                                            rem)], idxr)
            pltpu.sync_copy(h_sp.at[idxr], g0.at[pl.ds(0, rem)])
            pltpu.sync_copy(g0.at[pl.ds(0, rem)],
                            lx_hbm.at[pl.ds(ebase + nfull * CH, rem),
                                      pl.ds(core * hid, hid)])

    return setup(h, ef)


def _sc_prop(cur, dis, ef, n, e):
    ept = e // NTILE
    nfull = ept // CH
    rem = ept - nfull * CH
    npt = n // NTILE
    h2 = cur.shape[1]
    assert nfull % 2 == 0 and nfull >= 4

    mesh = plsc.VectorSubcoreMesh(core_axis_name="c", subcore_axis_name="s")

    @functools.partial(
        pl.kernel,
        mesh=mesh,
        compiler_params=_SC_PARAMS,
        out_type=jax.ShapeDtypeStruct((e, h2), jnp.float32),
        scratch_types=[
            pltpu.VMEM_SHARED((n, FH), jnp.float32),
            pltpu.VMEM((npt, FH), jnp.float32),
            pltpu.VMEM((CH, FH), jnp.float32),
            pltpu.VMEM((CH, FH), jnp.float32),
            pltpu.VMEM((CH, FH), jnp.float32),
            pltpu.VMEM((CH, FH), jnp.float32),
            pltpu.VMEM((CH, FH), jnp.float32),
            pltpu.VMEM((CH, FH), jnp.float32),
            pltpu.VMEM((CH,), jnp.float32),
            pltpu.VMEM((CH,), jnp.float32),
            pltpu.VMEM((CH,), jnp.int32),
            pltpu.VMEM((CH,), jnp.int32),
            pltpu.VMEM((16,), jnp.int32),
            pltpu.SemaphoreType.DMA,
            pltpu.SemaphoreType.DMA,
            pltpu.SemaphoreType.DMA,
            pltpu.SemaphoreType.DMA,
        ],
    )
    def prop(cur_hbm, dis_hbm, ef_hbm, p_hbm, agg_sp, zstage, z0, z1, u_f32,
             p0, p1, g_v, d0, d1, i0, i1, idxr, lsem0, lsem1, osem0, osem1):
        core = lax.axis_index("c")
        tid = lax.axis_index("s")
        ebase = tid * ept
        fbase = core * FH
        zvs, pvs, dvs, ivs = [z0, z1], [p0, p1], [d0, d1], [i0, i1]
        lsems, osems = [lsem0, lsem1], [osem0, osem1]

        @pl.loop(0, npt)
        def _z(r):
            for f in range(FH // 16):
                zstage[r, pl.ds(f * 16, 16)] = jnp.zeros((16,), jnp.float32)

        pltpu.sync_copy(zstage, agg_sp.at[pl.ds(tid * npt, npt)])
        plsc.subcore_barrier()

        def load_descs(k, b, use_dst):
            base = ebase + k * CH
            ioff = e + base if use_dst else base
            return (
                pltpu.make_async_copy(ef_hbm.at[pl.ds(ioff, CH)], ivs[b],
                                      lsems[b]),
                pltpu.make_async_copy(dis_hbm.at[pl.ds(base, CH)], dvs[b],
                                      lsems[b]),
                pltpu.make_async_copy(
                    cur_hbm.at[pl.ds(base, CH), pl.ds(fbase, FH)], zvs[b],
                    lsems[b]),
            )

        def scale_row(z_v, i, s):
            for f in range(FH // 16):
                sl = pl.ds(f * 16, 16)
                u_f32[i, sl] = s * z_v[i, sl]

        def combine_row(z_v, i, s, dst):
            s2 = s * s
            for f in range(FH // 16):
                sl = pl.ds(f * 16, 16)
                dst[i, sl] = s * g_v[i, sl] + s2 * z_v[i, sl]

        def scale_chunk(b, nrows):
            z_v, d_v = zvs[b], dvs[b]

            @pl.loop(0, nrows // 16)
            def _s(g):
                dvec = d_v[pl.ds(g * 16, 16)]
                for j in range(16):
                    scale_row(z_v, g * 16 + j, dvec[j])

        def combine_chunk(b, nrows, dst):
            z_v, d_v = zvs[b], dvs[b]

            @pl.loop(0, nrows // 16)
            def _c(g):
                dvec = d_v[pl.ds(g * 16, 16)]
                for j in range(16):
                    combine_row(z_v, g * 16 + j, dvec[j], dst)

        for b in range(2):
            for dsc in load_descs(b, b, True):
                dsc.start()

        @pl.loop(0, nfull // 2)
        def _scat(gi):
            for b in range(2):
                k = gi * 2 + b
                for dsc in load_descs(k, b, True):
                    dsc.wait()
                scale_chunk(b, CH)
                pltpu.sync_copy(u_f32, agg_sp.at[ivs[b]], add=True)

                @pl.when(k + 2 < nfull)
                def _():
                    for dsc in load_descs(k + 2, b, True):
                        dsc.start()

        if rem:
            base = ebase + nfull * CH
            pltpu.sync_copy(ef_hbm.at[pl.ds(e + base, rem)], idxr)
            pltpu.sync_copy(dis_hbm.at[pl.ds(base, rem)], d0.at[pl.ds(0, rem)])
            pltpu.sync_copy(cur_hbm.at[pl.ds(base, rem), pl.ds(fbase, FH)],
                            z0.at[pl.ds(0, rem)])

            @pl.loop(0, rem // 16)
            def _sr(g):
                dvec = d0[pl.ds(g * 16, 16)]
                for j in range(16):
                    scale_row(z0, g * 16 + j, dvec[j])

            pltpu.sync_copy(u_f32.at[pl.ds(0, rem)], agg_sp.at[idxr], add=True)
        plsc.subcore_barrier()

        def out_desc(k, b):
            base = ebase + k * CH
            return pltpu.make_async_copy(
                pvs[b], p_hbm.at[pl.ds(base, CH), pl.ds(fbase, FH)], osems[b])

        for b in range(2):
            for dsc in load_descs(b, b, False):
                dsc.start()

        @pl.loop(0, nfull // 2)
        def _gath(gi):
            for b in range(2):
                k = gi * 2 + b
                for dsc in load_descs(k, b, False):
                    dsc.wait()
                pltpu.sync_copy(agg_sp.at[ivs[b]], g_v)

                @pl.when(gi >= 1)
                def _():
                    out_desc(k, b).wait()

                combine_chunk(b, CH, pvs[b])
                out_desc(k, b).start()

                @pl.when(k + 2 < nfull)
                def _():
                    for dsc in load_descs(k + 2, b, False):
                        dsc.start()

        for b in range(2):
            out_desc(nfull - 2 + b, b).wait()

        if rem:
            base = ebase + nfull * CH
            pltpu.sync_copy(ef_hbm.at[pl.ds(base, rem)], idxr)
            pltpu.sync_copy(dis_hbm.at[pl.ds(base, rem)], d0.at[pl.ds(0, rem)])
            pltpu.sync_copy(cur_hbm.at[pl.ds(base, rem), pl.ds(fbase, FH)],
                            z0.at[pl.ds(0, rem)])
            pltpu.sync_copy(agg_sp.at[idxr], g_v.at[pl.ds(0, rem)])

            @pl.loop(0, rem // 16)
            def _cr(g):
                dvec = d0[pl.ds(g * 16, 16)]
                for j in range(16):
                    combine_row(z0, g * 16 + j, dvec[j], p0)

            pltpu.sync_copy(p0.at[pl.ds(0, rem)],
                            p_hbm.at[pl.ds(base, rem), pl.ds(fbase, FH)])

    return prop(cur, dis, ef)


def kernel(x, edge_index, W0, b0, conv_W, W_out, b_out):
    n = x.shape[0]
    e = edge_index.shape[1]
    num_layers = conv_W.shape[0]

    ef = edge_index.reshape(-1).astype(jnp.int32)

    h = _lin0(x, W0, b0)
    lx, dis = _sc_setup(h, ef, n, e)

    cur = lx
    out = None
    for l in range(num_layers):
        beta = float(np.log(THETA / (l + 1) + 1.0))
        p = _sc_prop(cur, dis, ef, n, e)
        last = l == num_layers - 1
        if last:
            out = _tc_layer(p, lx, conv_W[l], beta, True, W_out, b_out)
        else:
            cur = _tc_layer(p, lx, conv_W[l], beta, False)
    return out


# speedup vs baseline: 1.8915x; 1.0288x over previous
import functools

import numpy as np
import jax
import jax.numpy as jnp
from jax import lax
from jax.experimental import pallas as pl
from jax.experimental.pallas import tpu as pltpu
from jax.experimental.pallas import tpu_sc as plsc

ALPHA = 0.1
THETA = 0.5

NTILE = 16
CH = 128
FH = 64

_SC_PARAMS = pltpu.CompilerParams(use_tc_tiling_on_sc=False,
                                  needs_layout_passes=False)


def _rsqrt16(x):
    i = plsc.bitcast(x, jnp.int32)
    i = jnp.int32(0x5F3759DF) - lax.shift_right_logical(i, 1)
    y = plsc.bitcast(i, jnp.float32)
    for _ in range(3):
        y = y * (1.5 - 0.5 * x * y * y)
    return y


def _lin0_body(x_ref, w_ref, b_ref, o_ref):
    acc = jnp.dot(x_ref[...], w_ref[...], preferred_element_type=jnp.float32)
    o_ref[...] = jnp.maximum(acc + b_ref[...], 0.0)


def _lin0(x, W0, b0):
    n, in_f = x.shape
    hid = W0.shape[1]
    bn = 2000
    grid = n // bn
    return pl.pallas_call(
        _lin0_body,
        grid=(grid,),
        in_specs=[
            pl.BlockSpec((bn, in_f), lambda i: (i, 0)),
            pl.BlockSpec((in_f, hid), lambda i: (0, 0)),
            pl.BlockSpec((1, hid), lambda i: (0, 0)),
        ],
        out_specs=pl.BlockSpec((bn, hid), lambda i: (i, 0)),
        out_shape=jax.ShapeDtypeStruct((n, hid), jnp.float32),
    )(x, W0, b0.reshape(1, hid))


def _make_mix(beta, last):
    a1 = 1.0 - ALPHA
    a0 = ALPHA
    b1 = 1.0 - beta
    b0c = beta

    def tmix(p_ref, x0_ref, w_ref):
        t = a1 * p_ref[...] + a0 * x0_ref[...]
        return b1 * t + b0c * jnp.dot(t, w_ref[...],
                                      preferred_element_type=jnp.float32)

    if last:
        def body(p_ref, x0_ref, w_ref, wo_ref, bo_ref, o_ref):
            r = jnp.maximum(tmix(p_ref, x0_ref, w_ref), 0.0)
            o_ref[...] = jnp.dot(r, wo_ref[...],
                                 preferred_element_type=jnp.float32) + bo_ref[...]
    else:
        def body(p_ref, x0_ref, w_ref, o_ref):
            r = jnp.maximum(tmix(p_ref, x0_ref, w_ref), 0.0)
            o_ref[...] = r
    return body


def _tc_layer(p, x0, Wl, beta, last, W_out=None, b_out=None):
    e, h2 = p.shape
    bn = 1280
    grid = e // bn
    out_f = W_out.shape[1] if last else h2
    blk = lambda i: (i, 0)
    zero = lambda i: (0, 0)
    in_specs = [
        pl.BlockSpec((bn, h2), blk),
        pl.BlockSpec((bn, h2), blk),
        pl.BlockSpec((h2, h2), zero),
    ]
    args = [p, x0, Wl]
    if last:
        in_specs += [pl.BlockSpec((h2, out_f), zero),
                     pl.BlockSpec((1, out_f), zero)]
        args += [W_out, b_out.reshape(1, out_f)]
    return pl.pallas_call(
        _make_mix(beta, last),
        grid=(grid,),
        in_specs=in_specs,
        out_specs=pl.BlockSpec((bn, out_f), blk),
        out_shape=jax.ShapeDtypeStruct((e, out_f), jnp.float32),
    )(*args)


def _sc_setup(h, ef, n, e):
    ept = e // NTILE
    nfull = ept // CH
    rem = ept - nfull * CH
    npt = n // NTILE
    hid = h.shape[1]
    assert nfull % 2 == 0 and nfull >= 4

    mesh = plsc.VectorSubcoreMesh(core_axis_name="c", subcore_axis_name="s")

    @functools.partial(
        pl.kernel,
        mesh=mesh,
        compiler_params=_SC_PARAMS,
        out_type=[
            jax.ShapeDtypeStruct((e, 2 * hid), jnp.float32),
            jax.ShapeDtypeStruct((e,), jnp.float32),
        ],
        scratch_types=[
            pltpu.VMEM_SHARED((n, hid), jnp.float32),
            pltpu.VMEM_SHARED((n, 16), jnp.float32),
            pltpu.VMEM((npt, hid), jnp.float32),
            pltpu.VMEM((CH, 16), jnp.float32),
            pltpu.VMEM((CH,), jnp.int32),
            pltpu.VMEM((CH,), jnp.int32),
            pltpu.VMEM((16,), jnp.int32),
            pltpu.VMEM((CH, 16), jnp.float32),
            pltpu.VMEM((ept,), jnp.float32),
            pltpu.VMEM((CH, hid), jnp.float32),
            pltpu.VMEM((CH, hid), jnp.float32),
            pltpu.SemaphoreType.DMA,
            pltpu.SemaphoreType.DMA,
            pltpu.SemaphoreType.DMA,
            pltpu.SemaphoreType.DMA,
        ],
    )
    def setup(h_hbm, ef_hbm, lx_hbm, dis_hbm, h_sp, deg_sp, stage, ones_v,
              i0, i1, idxr, dtmp, dis_all, g0, g1,
              lsem0, lsem1, osem0, osem1):
        core = lax.axis_index("c")
        tid = lax.axis_index("s")
        ebase = tid * ept
        ivs, gvs = [i0, i1], [g0, g1]
        lsems, osems = [lsem0, lsem1], [osem0, osem1]

        @pl.loop(0, npt)
        def _z(r):
            for f in range(hid // 16):
                stage[r, pl.ds(f * 16, 16)] = jnp.zeros((16,), jnp.float32)

        pltpu.sync_copy(stage.at[:, pl.ds(0, 16)],
                        deg_sp.at[pl.ds(tid * npt, npt)])
        pltpu.sync_copy(h_hbm.at[pl.ds(tid * npt, npt)], stage)
        pltpu.sync_copy(stage, h_sp.at[pl.ds(tid * npt, npt)])
        plsc.subcore_barrier()

        @pl.loop(0, CH)
        def _o(r):
            ones_v[r] = jnp.ones((16,), jnp.float32)

        def idx_desc(k, b, which):
            return pltpu.make_async_copy(
                ef_hbm.at[pl.ds(which * e + ebase + k * CH, CH)], ivs[b],
                lsems[b])

        for b in range(2):
            idx_desc(b, b, 1).start()

        @pl.loop(0, nfull // 2)
        def _deg(gi):
            for b in range(2):
                k = gi * 2 + b
                idx_desc(k, b, 1).wait()
                pltpu.sync_copy(ones_v, deg_sp.at[ivs[b]], add=True)

                @pl.when(k + 2 < nfull)
                def _():
                    idx_desc(k + 2, b, 1).start()

        if rem:
            pltpu.sync_copy(ef_hbm.at[pl.ds(e + ebase + nfull * CH, rem)],
                            idxr)
            pltpu.sync_copy(ones_v.at[pl.ds(0, rem)], deg_sp.at[idxr],
                            add=True)
        plsc.subcore_barrier()

        lanes = lax.iota(jnp.int32, 16)
        zeros16 = jnp.zeros((16,), jnp.int32)

        for b in range(2):
            idx_desc(b, b, 0).start()

        @pl.loop(0, nfull // 2)
        def _dis(gi):
            for b in range(2):
                k = gi * 2 + b
                idx_desc(k, b, 0).wait()
                pltpu.sync_copy(deg_sp.at[ivs[b]], dtmp)
                for j in range(CH // 16):
                    d = plsc.load_gather(dtmp, [j * 16 + lanes, zeros16])
                    dis_all[pl.ds(k * CH + j * 16, 16)] = _rsqrt16(d + 1.0)

                @pl.when(k + 2 < nfull)
                def _():
                    idx_desc(k + 2, b, 0).start()

        if rem:
            pltpu.sync_copy(ef_hbm.at[pl.ds(ebase + nfull * CH, rem)], idxr)
            pltpu.sync_copy(deg_sp.at[idxr], dtmp.at[pl.ds(0, rem)])
            for j in range(rem // 16):
                d = plsc.load_gather(dtmp, [j * 16 + lanes, zeros16])
                dis_all[pl.ds(nfull * CH + j * 16, 16)] = _rsqrt16(d + 1.0)

        @pl.when(core == 0)
        def _():
            pltpu.sync_copy(dis_all, dis_hbm.at[pl.ds(ebase, ept)])

        def out_desc(k, b):
            return pltpu.make_async_copy(
                gvs[b],
                lx_hbm.at[pl.ds(ebase + k * CH, CH), pl.ds(core * hid, hid)],
                osems[b])

        for b in range(2):
            idx_desc(b, b, core).start()

        @pl.loop(0, nfull // 2)
        def _lx(gi):
            for b in range(2):
                k = gi * 2 + b
                idx_desc(k, b, core).wait()

                @pl.when(gi >= 1)
                def _():
                    out_desc(k, b).wait()

                pltpu.sync_copy(h_sp.at[ivs[b]], gvs[b])
                out_desc(k, b).start()

                @pl.when(k + 2 < nfull)
                def _():
                    idx_desc(k + 2, b, core).start()

        for b in range(2):
            out_desc(nfull - 2 + b, b).wait()

        if rem:
            pltpu.sync_copy(ef_hbm.at[pl.ds(core * e + ebase + nfull * CH,
                                            rem)], idxr)
            pltpu.sync_copy(h_sp.at[idxr], g0.at[pl.ds(0, rem)])
            pltpu.sync_copy(g0.at[pl.ds(0, rem)],
                            lx_hbm.at[pl.ds(ebase + nfull * CH, rem),
                                      pl.ds(core * hid, hid)])

    return setup(h, ef)


def _sc_prop(cur, dis, ef, n, e):
    ept = e // NTILE
    nfull = ept // CH
    rem = ept - nfull * CH
    npt = n // NTILE
    h2 = cur.shape[1]
    assert nfull % 2 == 0 and nfull >= 4

    mesh = plsc.VectorSubcoreMesh(core_axis_name="c", subcore_axis_name="s")

    @functools.partial(
        pl.kernel,
        mesh=mesh,
        compiler_params=_SC_PARAMS,
        out_type=jax.ShapeDtypeStruct((e, h2), jnp.float32),
        scratch_types=[
            pltpu.VMEM_SHARED((n, FH), jnp.float32),
            pltpu.VMEM((npt, FH), jnp.float32),
            pltpu.VMEM((CH, FH), jnp.float32),
            pltpu.VMEM((CH, FH), jnp.float32),
            pltpu.VMEM((CH, FH), jnp.float32),
            pltpu.VMEM((CH, FH), jnp.float32),
            pltpu.VMEM((CH, FH), jnp.float32),
            pltpu.VMEM((CH, FH), jnp.float32),
            pltpu.VMEM((CH,), jnp.float32),
            pltpu.VMEM((CH,), jnp.float32),
            pltpu.VMEM((CH,), jnp.int32),
            pltpu.VMEM((CH,), jnp.int32),
            pltpu.VMEM((16,), jnp.int32),
            pltpu.VMEM((CH,), jnp.int32),
            pltpu.VMEM((CH,), jnp.int32),
            pltpu.SemaphoreType.DMA,
            pltpu.SemaphoreType.DMA,
            pltpu.SemaphoreType.DMA,
            pltpu.SemaphoreType.DMA,
            pltpu.SemaphoreType.DMA,
            pltpu.SemaphoreType.DMA,
        ],
    )
    def prop(cur_hbm, dis_hbm, ef_hbm, p_hbm, agg_sp, zstage, z0, z1, u_f32,
             p0, p1, g_v, d0, d1, i0, i1, idxr, si0, si1,
             lsem0, lsem1, osem0, osem1, ssem0, ssem1):
        core = lax.axis_index("c")
        tid = lax.axis_index("s")
        ebase = tid * ept
        fbase = core * FH
        zvs, pvs, dvs, ivs = [z0, z1], [p0, p1], [d0, d1], [i0, i1]
        sis = [si0, si1]
        uvs = [u_f32, g_v]
        lsems, osems = [lsem0, lsem1], [osem0, osem1]
        ssems = [ssem0, ssem1]

        @pl.loop(0, npt)
        def _z(r):
            for f in range(FH // 16):
                zstage[r, pl.ds(f * 16, 16)] = jnp.zeros((16,), jnp.float32)

        pltpu.sync_copy(zstage, agg_sp.at[pl.ds(tid * npt, npt)])
        plsc.subcore_barrier()

        def load_descs(k, b, use_dst):
            base = ebase + k * CH
            ioff = e + base if use_dst else base
            return (
                pltpu.make_async_copy(ef_hbm.at[pl.ds(ioff, CH)], ivs[b],
                                      lsems[b]),
                pltpu.make_async_copy(dis_hbm.at[pl.ds(base, CH)], dvs[b],
                                      lsems[b]),
                pltpu.make_async_copy(
                    cur_hbm.at[pl.ds(base, CH), pl.ds(fbase, FH)], zvs[b],
                    lsems[b]),
            )

        def scale_row(z_v, i, s, dst):
            for f in range(FH // 16):
                sl = pl.ds(f * 16, 16)
                dst[i, sl] = s * z_v[i, sl]

        def combine_row(z_v, i, s, dst):
            s2 = s * s
            for f in range(FH // 16):
                sl = pl.ds(f * 16, 16)
                dst[i, sl] = s * g_v[i, sl] + s2 * z_v[i, sl]

        def scale_chunk(b, nrows, dst):
            z_v, d_v = zvs[b], dvs[b]

            @pl.loop(0, nrows // 16)
            def _s(g):
                dvec = d_v[pl.ds(g * 16, 16)]
                for j in range(16):
                    scale_row(z_v, g * 16 + j, dvec[j], dst)

        def combine_chunk(b, nrows, dst):
            z_v, d_v = zvs[b], dvs[b]

            @pl.loop(0, nrows // 16)
            def _c(g):
                dvec = d_v[pl.ds(g * 16, 16)]
                for j in range(16):
                    combine_row(z_v, g * 16 + j, dvec[j], dst)

        def scat_start(b):
            pltpu.async_copy(uvs[b], agg_sp.at[sis[b]], ssems[b], add=True)

        def scat_wait(b):
            pltpu.make_async_copy(uvs[b], agg_sp.at[sis[b]], ssems[b]).wait()

        def copy_idx(b):
            @pl.loop(0, CH // 16)
            def _ci(g):
                sl = pl.ds(g * 16, 16)
                sis[b][sl] = ivs[b][sl]

        for b in range(2):
            for dsc in load_descs(b, b, True):
                dsc.start()

        @pl.loop(0, nfull // 2)
        def _scat(gi):
            for b in range(2):
                k = gi * 2 + b
                for dsc in load_descs(k, b, True):
                    dsc.wait()

                @pl.when(gi >= 1)
                def _():
                    scat_wait(b)

                copy_idx(b)
                scale_chunk(b, CH, uvs[b])
                scat_start(b)

                @pl.when(k + 2 < nfull)
                def _():
                    for dsc in load_descs(k + 2, b, True):
                        dsc.start()

        for b in range(2):
            scat_wait(b)

        if rem:
            base = ebase + nfull * CH
            pltpu.sync_copy(ef_hbm.at[pl.ds(e + base, rem)], idxr)
            pltpu.sync_copy(dis_hbm.at[pl.ds(base, rem)], d0.at[pl.ds(0, rem)])
            pltpu.sync_copy(cur_hbm.at[pl.ds(base, rem), pl.ds(fbase, FH)],
                            z0.at[pl.ds(0, rem)])

            @pl.loop(0, rem // 16)
            def _sr(g):
                dvec = d0[pl.ds(g * 16, 16)]
                for j in range(16):
                    scale_row(z0, g * 16 + j, dvec[j], u_f32)

            pltpu.sync_copy(u_f32.at[pl.ds(0, rem)], agg_sp.at[idxr], add=True)
        plsc.subcore_barrier()

        def out_desc(k, b):
            base = ebase + k * CH
            return pltpu.make_async_copy(
                pvs[b], p_hbm.at[pl.ds(base, CH), pl.ds(fbase, FH)], osems[b])

        for b in range(2):
            for dsc in load_descs(b, b, False):
                dsc.start()

        @pl.loop(0, nfull // 2)
        def _gath(gi):
            for b in range(2):
                k = gi * 2 + b
                for dsc in load_descs(k, b, False):
                    dsc.wait()
                pltpu.sync_copy(agg_sp.at[ivs[b]], g_v)

                @pl.when(gi >= 1)
                def _():
                    out_desc(k, b).wait()

                combine_chunk(b, CH, pvs[b])
                out_desc(k, b).start()

                @pl.when(k + 2 < nfull)
                def _():
                    for dsc in load_descs(k + 2, b, False):
                        dsc.start()

        for b in range(2):
            out_desc(nfull - 2 + b, b).wait()

        if rem:
            base = ebase + nfull * CH
            pltpu.sync_copy(ef_hbm.at[pl.ds(base, rem)], idxr)
            pltpu.sync_copy(dis_hbm.at[pl.ds(base, rem)], d0.at[pl.ds(0, rem)])
            pltpu.sync_copy(cur_hbm.at[pl.ds(base, rem), pl.ds(fbase, FH)],
                            z0.at[pl.ds(0, rem)])
            pltpu.sync_copy(agg_sp.at[idxr], g_v.at[pl.ds(0, rem)])

            @pl.loop(0, rem // 16)
            def _cr(g):
                dvec = d0[pl.ds(g * 16, 16)]
                for j in range(16):
                    combine_row(z0, g * 16 + j, dvec[j], p0)

            pltpu.sync_copy(p0.at[pl.ds(0, rem)],
                            p_hbm.at[pl.ds(base, rem), pl.ds(fbase, FH)])

    return prop(cur, dis, ef)


def kernel(x, edge_index, W0, b0, conv_W, W_out, b_out):
    n = x.shape[0]
    e = edge_index.shape[1]
    num_layers = conv_W.shape[0]

    ef = edge_index.reshape(-1).astype(jnp.int32)

    h = _lin0(x, W0, b0)
    lx, dis = _sc_setup(h, ef, n, e)

    cur = lx
    out = None
    for l in range(num_layers):
        beta = float(np.log(THETA / (l + 1) + 1.0))
        p = _sc_prop(cur, dis, ef, n, e)
        last = l == num_layers - 1
        if last:
            out = _tc_layer(p, lx, conv_W[l], beta, True, W_out, b_out)
        else:
            cur = _tc_layer(p, lx, conv_W[l], beta, False)
    return out
